# baseline (speedup 1.0000x reference)
import jax
import jax.numpy as jnp
from jax.experimental import pallas as pl
from jax.experimental.pallas import tpu as pltpu

N = 2048
D = 768
H = 768
E = 8
BN = 256
NB = N // BN


def _moe_body(x_ref, wg_ref, bg_ref, w1_ref, b1_ref, g1_ref, be1_ref,
              w2_ref, b2_ref, out_ref, w1s, w2s):
    s = pl.program_id(0)

    @pl.when(s < E)
    def _():
        e = jnp.minimum(s, E - 1)
        w1s[e] = w1_ref[0].astype(jnp.bfloat16)
        w2s[pl.ds(e * H, H), :] = w2_ref[0].astype(jnp.bfloat16)

    @pl.when(s >= E)
    def _():
        xb = x_ref[...]

        logits = jnp.dot(xb, wg_ref[...], preferred_element_type=jnp.float32)
        logits = logits + bg_ref[...]
        neg_inf = jnp.float32(-jnp.inf)
        iota = jax.lax.broadcasted_iota(jnp.int32, logits.shape, 1)
        m1 = jnp.max(logits, axis=-1, keepdims=True)
        eq1 = logits == m1
        i1 = jnp.min(jnp.where(eq1, iota, E), axis=-1, keepdims=True)
        first1 = iota == i1
        l2 = jnp.where(first1, neg_inf, logits)
        m2 = jnp.max(l2, axis=-1, keepdims=True)
        eq2 = l2 == m2
        i2 = jnp.min(jnp.where(eq2, iota, E), axis=-1, keepdims=True)
        first2 = iota == i2
        sel = first1 | first2
        wsel = jnp.where(sel, jnp.exp(logits - m1), 0.0)
        cw = wsel / jnp.sum(wsel, axis=-1, keepdims=True)

        xb16 = xb.astype(jnp.bfloat16)
        parts = []
        for e in range(E):
            h = jnp.dot(xb16, w1s[e], preferred_element_type=jnp.float32)
            h = h + b1_ref[e][None, :]
            mu = jnp.mean(h, axis=-1, keepdims=True)
            var = jnp.mean((h - mu) ** 2, axis=-1, keepdims=True)
            h = (h - mu) * jax.lax.rsqrt(var + 1e-5)
            h = h * g1_ref[e][None, :] + be1_ref[e][None, :]
            h = h * 0.5 * (1.0 + jax.lax.erf(h * jnp.float32(0.7071067811865476)))
            parts.append((h * cw[:, e][:, None]).astype(jnp.bfloat16))
        hcat = jnp.concatenate(parts, axis=1)
        out = jnp.dot(hcat, w2s[...], preferred_element_type=jnp.float32)
        out_ref[...] = out + jnp.dot(cw, b2_ref[...],
                                     preferred_element_type=jnp.float32)


def kernel(x, Wg, bg, W1, b1, g1, be1, W2, b2):
    const = lambda s: (0, 0)

    def wmap(s):
        return (jnp.minimum(s, E - 1), 0, 0)

    def xmap(s):
        return (jnp.maximum(s - E, 0), 0)

    out = pl.pallas_call(
        _moe_body,
        grid=(E + NB,),
        in_specs=[
            pl.BlockSpec((BN, D), xmap),
            pl.BlockSpec((D, E), const),
            pl.BlockSpec((1, E), const),
            pl.BlockSpec((1, D, H), wmap),
            pl.BlockSpec((E, H), const),
            pl.BlockSpec((E, H), const),
            pl.BlockSpec((E, H), const),
            pl.BlockSpec((1, H, H), wmap),
            pl.BlockSpec((E, H), const),
        ],
        out_specs=pl.BlockSpec((BN, H), xmap),
        out_shape=jax.ShapeDtypeStruct((N, H), jnp.float32),
        scratch_shapes=[
            pltpu.VMEM((E, D, H), jnp.bfloat16),
            pltpu.VMEM((E * H, H), jnp.bfloat16),
        ],
    )(x, Wg, bg.reshape(1, E), W1, b1, g1, be1, W2, b2)
    return out


# speedup vs baseline: 1.0151x; 1.0151x over previous
import jax
import jax.numpy as jnp
from jax.experimental import pallas as pl
from jax.experimental.pallas import tpu as pltpu

N = 2048
D = 768
H = 768
E = 8
BN = 512
NB = N // BN


def _moe_body(x_ref, wg_ref, bg_ref, w1_ref, b1_ref, g1_ref, be1_ref,
              w2_ref, b2_ref, out_ref, w1s, w2s):
    s = pl.program_id(0)

    @pl.when(s < E)
    def _():
        e = jnp.minimum(s, E - 1)
        w1s[e] = w1_ref[0].astype(jnp.bfloat16)
        w2s[pl.ds(e * H, H), :] = w2_ref[0].astype(jnp.bfloat16)

    @pl.when(s >= E)
    def _():
        xb = x_ref[...]

        logits = jnp.dot(xb, wg_ref[...], preferred_element_type=jnp.float32)
        logits = logits + bg_ref[...]
        neg_inf = jnp.float32(-jnp.inf)
        iota = jax.lax.broadcasted_iota(jnp.int32, logits.shape, 1)
        m1 = jnp.max(logits, axis=-1, keepdims=True)
        eq1 = logits == m1
        i1 = jnp.min(jnp.where(eq1, iota, E), axis=-1, keepdims=True)
        first1 = iota == i1
        l2 = jnp.where(first1, neg_inf, logits)
        m2 = jnp.max(l2, axis=-1, keepdims=True)
        eq2 = l2 == m2
        i2 = jnp.min(jnp.where(eq2, iota, E), axis=-1, keepdims=True)
        first2 = iota == i2
        sel = first1 | first2
        wsel = jnp.where(sel, jnp.exp(logits - m1), 0.0)
        cw = wsel / jnp.sum(wsel, axis=-1, keepdims=True)

        xb16 = xb.astype(jnp.bfloat16)
        parts = []
        for e in range(E):
            h = jnp.dot(xb16, w1s[e], preferred_element_type=jnp.float32)
            h = h + b1_ref[e][None, :]
            mu = jnp.mean(h, axis=-1, keepdims=True)
            var = jnp.mean((h - mu) ** 2, axis=-1, keepdims=True)
            h = (h - mu) * jax.lax.rsqrt(var + 1e-5)
            h = h * g1_ref[e][None, :] + be1_ref[e][None, :]
            h = h * 0.5 * (1.0 + jax.lax.erf(h * jnp.float32(0.7071067811865476)))
            parts.append((h * cw[:, e][:, None]).astype(jnp.bfloat16))
        hcat = jnp.concatenate(parts, axis=1)
        out = jnp.dot(hcat, w2s[...], preferred_element_type=jnp.float32)
        out_ref[...] = out + jnp.dot(cw, b2_ref[...],
                                     preferred_element_type=jnp.float32)


def kernel(x, Wg, bg, W1, b1, g1, be1, W2, b2):
    const = lambda s: (0, 0)

    def wmap(s):
        return (jnp.minimum(s, E - 1), 0, 0)

    def xmap(s):
        return (jnp.maximum(s - E, 0), 0)

    out = pl.pallas_call(
        _moe_body,
        grid=(E + NB,),
        in_specs=[
            pl.BlockSpec((BN, D), xmap),
            pl.BlockSpec((D, E), const),
            pl.BlockSpec((1, E), const),
            pl.BlockSpec((1, D, H), wmap),
            pl.BlockSpec((E, H), const),
            pl.BlockSpec((E, H), const),
            pl.BlockSpec((E, H), const),
            pl.BlockSpec((1, H, H), wmap),
            pl.BlockSpec((E, H), const),
        ],
        out_specs=pl.BlockSpec((BN, H), xmap),
        out_shape=jax.ShapeDtypeStruct((N, H), jnp.float32),
        scratch_shapes=[
            pltpu.VMEM((E, D, H), jnp.bfloat16),
            pltpu.VMEM((E * H, H), jnp.bfloat16),
        ],
    )(x, Wg, bg.reshape(1, E), W1, b1, g1, be1, W2, b2)
    return out
